# baseline (speedup 1.0000x reference)
import jax
import jax.numpy as jnp
from jax import lax
from jax.experimental import pallas as pl
from jax.experimental.pallas import tpu as pltpu
from jax.experimental.pallas import tpu_sc as plsc

_BATCH = 16384
_NUM_FIELDS = 26
_VOCAB = 100000
_EMB_DIM = 32

_NC = 2
_NS = 16
_C = 50048
_CB = _VOCAB - _C
_Q = 4096
_NQ = _BATCH // _Q


def _sc_body(x_hbm, table_hbm, out_hbm, buf_a, buf_b, idx0, idx1, acc,
             sem_a, sem_b, sem_i0, sem_i1):
    s = lax.axis_index("s") * _NC + lax.axis_index("c")
    idx_bufs = (idx0, idx1)
    idx_sems = (sem_i0, sem_i1)

    def stage_a(i):
        return pltpu.async_copy(table_hbm.at[i, s, pl.ds(0, _C)], buf_a, sem_a)

    def stage_b(i):
        return pltpu.async_copy(table_hbm.at[i, s, pl.ds(_C, _CB)], buf_b, sem_b)

    def stage_idx(i, q, slot):
        return pltpu.async_copy(
            x_hbm.at[i, pl.ds(q * _Q, _Q)], idx_bufs[slot], idx_sems[slot])

    def drain_idx(slot):
        pltpu.make_async_copy(
            x_hbm.at[0, pl.ds(0, _Q)], idx_bufs[slot], idx_sems[slot]).wait()

    def run_pass(i, in_a, first):
        stage_idx(i, 0, 0)

        def quarter(q, slot):
            drain_idx(slot)

            @pl.when(q + 1 < _NQ)
            def _():
                stage_idx(i, q + 1, 1 - slot)

            base = q * _Q

            def chunk(j, carry):
                v = idx_bufs[slot][pl.ds(j * 16, 16)]
                if in_a:
                    va = jnp.minimum(v, _C - 1)
                    g = plsc.load_gather(buf_a, [va])
                    m = v < _C
                else:
                    vb = jnp.maximum(v - _C, 0)
                    g = plsc.load_gather(buf_b, [vb])
                    m = v >= _C
                g = jnp.where(m, g, jnp.zeros((16,), jnp.float32))
                sl = pl.ds(base + j * 16, 16)
                if first:
                    acc[sl] = g
                else:
                    plsc.addupdate(acc.at[sl], g)
                return carry
            lax.fori_loop(0, _Q // 16, chunk, 0, unroll=8)

        for q in range(_NQ):
            quarter(q, q % 2)

    def drain_a():
        pltpu.make_async_copy(
            table_hbm.at[0, 0, pl.ds(0, _C)], buf_a, sem_a).wait()

    def drain_b():
        pltpu.make_async_copy(
            table_hbm.at[0, 0, pl.ds(_C, _CB)], buf_b, sem_b).wait()

    stage_a(0)
    stage_b(0)

    def field(i, carry):
        nxt = jnp.minimum(i + 1, _NUM_FIELDS - 1)
        drain_a()
        run_pass(i, True, False)
        stage_a(nxt)
        drain_b()
        run_pass(i, False, False)
        stage_b(nxt)
        return carry

    drain_a()
    run_pass(0, True, True)
    stage_a(1)
    drain_b()
    run_pass(0, False, False)
    stage_b(1)
    lax.fori_loop(1, _NUM_FIELDS, field, 0)
    drain_a()
    drain_b()
    pltpu.sync_copy(acc, out_hbm.at[s])


def kernel(x, tables):
    x_t = x.T
    tables_t = tables.transpose(0, 2, 1)

    f = pl.kernel(
        _sc_body,
        out_type=jax.ShapeDtypeStruct((_EMB_DIM, _BATCH), jnp.float32),
        mesh=plsc.VectorSubcoreMesh(core_axis_name="c", subcore_axis_name="s"),
        scratch_types=[
            pltpu.VMEM((_C,), jnp.float32),
            pltpu.VMEM((_CB,), jnp.float32),
            pltpu.VMEM((_Q,), jnp.int32),
            pltpu.VMEM((_Q,), jnp.int32),
            pltpu.VMEM((_BATCH,), jnp.float32),
            pltpu.SemaphoreType.DMA,
            pltpu.SemaphoreType.DMA,
            pltpu.SemaphoreType.DMA,
            pltpu.SemaphoreType.DMA,
        ],
        compiler_params=pltpu.CompilerParams(
            use_tc_tiling_on_sc=True, needs_layout_passes=False),
    )
    return f(x_t, tables_t).T


# speedup vs baseline: 1.2436x; 1.2436x over previous
import jax
import jax.numpy as jnp
from jax import lax
from jax.experimental import pallas as pl
from jax.experimental.pallas import tpu as pltpu
from jax.experimental.pallas import tpu_sc as plsc

_BATCH = 16384
_NUM_FIELDS = 26
_VOCAB = 100000
_EMB_DIM = 32

_NC = 2
_C = 50048
_CB = _VOCAB - _C
_Q = 4096
_NQ = _BATCH // _Q


def _sc_body(x_hbm, table_hbm, out_hbm, buf_a, buf_b, buf_t, idx0, idx1,
             acc, idx_s, sem_a, sem_b, sem_t, sem_i0, sem_i1, sem_s):
    t = lax.axis_index("s")
    c = lax.axis_index("c")
    s = t * _NC + c
    idx_bufs = (idx0, idx1)
    idx_sems = (sem_i0, sem_i1)
    zeros16 = jnp.zeros((16,), jnp.float32)

    def stage_a(i):
        pltpu.async_copy(
            table_hbm.at[i, s, pl.ds(0, _C)], buf_a.at[pl.ds(0, _C)], sem_a)

    def stage_b(i):
        pltpu.async_copy(
            table_hbm.at[i, s, pl.ds(_C, _CB - 32)],
            buf_b.at[pl.ds(0, _CB - 32)], sem_b)
        pltpu.async_copy(
            table_hbm.at[i, s, pl.ds(_VOCAB - 32, 32)], buf_t, sem_t)

    def drain_a():
        pltpu.make_async_copy(
            table_hbm.at[0, 0, pl.ds(0, _C)], buf_a.at[pl.ds(0, _C)],
            sem_a).wait()

    def drain_b():
        pltpu.make_async_copy(
            table_hbm.at[0, 0, pl.ds(_C, _CB - 32)],
            buf_b.at[pl.ds(0, _CB - 32)], sem_b).wait()
        pltpu.make_async_copy(
            table_hbm.at[0, 0, pl.ds(_VOCAB - 32, 32)], buf_t, sem_t).wait()
        buf_b[pl.ds(_CB - 32, 16)] = buf_t[pl.ds(0, 16)]
        buf_b[pl.ds(_CB - 16, 16)] = buf_t[pl.ds(16, 16)]

    def stage_idx_hbm(i, slot):
        pltpu.async_copy(x_hbm.at[i, :], idx_s.at[slot], sem_s)

    def wait_idx_hbm(slot):
        pltpu.make_async_copy(x_hbm.at[0, :], idx_s.at[slot], sem_s).wait()

    def stage_q(sslot, q, slot):
        pltpu.async_copy(
            idx_s.at[sslot, pl.ds(q * _Q, _Q)], idx_bufs[slot],
            idx_sems[slot])

    def drain_q(sslot, slot):
        pltpu.make_async_copy(
            idx_s.at[sslot, pl.ds(0, _Q)], idx_bufs[slot],
            idx_sems[slot]).wait()

    def run_pass(sslot, in_a, first):
        stage_q(sslot, 0, 0)
        for q in range(_NQ):
            slot = q % 2
            drain_q(sslot, slot)
            if q + 1 < _NQ:
                stage_q(sslot, q + 1, 1 - slot)
            base = q * _Q

            def chunk(j, carry):
                v = idx_bufs[slot][pl.ds(j * 16, 16)]
                if in_a:
                    g = plsc.load_gather(buf_a, [jnp.minimum(v, _C)])
                else:
                    vb = plsc.bitcast(
                        jnp.minimum(
                            plsc.bitcast(v - _C, jnp.uint32),
                            jnp.uint32(_CB)),
                        jnp.int32)
                    g = plsc.load_gather(buf_b, [vb])
                sl = pl.ds(base + j * 16, 16)
                if first:
                    acc[sl] = g
                else:
                    plsc.addupdate(acc.at[sl], g)
                return carry
            lax.fori_loop(0, _Q // 16, chunk, 0, unroll=8)

    buf_a[pl.ds(_C, 16)] = zeros16
    buf_b[pl.ds(_CB, 16)] = zeros16

    @pl.when(t == 0)
    def _():
        stage_idx_hbm(0, 0)
        wait_idx_hbm(0)
    stage_a(0)
    stage_b(0)
    plsc.subcore_barrier()

    def field(i, first):
        sslot = lax.rem(i, 2)
        nxt = jnp.minimum(i + 1, _NUM_FIELDS - 1)

        @pl.when(t == 0)
        def _():
            stage_idx_hbm(nxt, 1 - sslot)

        drain_a()
        run_pass(sslot, True, first)
        stage_a(nxt)
        drain_b()
        run_pass(sslot, False, False)
        stage_b(nxt)

        @pl.when(t == 0)
        def _():
            wait_idx_hbm(1 - sslot)
        plsc.subcore_barrier()

    field(0, True)
    lax.fori_loop(1, _NUM_FIELDS, lambda i, cr: (field(i, False), cr)[1], 0)
    drain_a()
    drain_b()
    pltpu.sync_copy(acc, out_hbm.at[s])


def kernel(x, tables):
    x_t = x.T
    tables_t = tables.transpose(0, 2, 1)

    f = pl.kernel(
        _sc_body,
        out_type=jax.ShapeDtypeStruct((_EMB_DIM, _BATCH), jnp.float32),
        mesh=plsc.VectorSubcoreMesh(core_axis_name="c", subcore_axis_name="s"),
        scratch_types=[
            pltpu.VMEM((_C + 16,), jnp.float32),
            pltpu.VMEM((_CB + 16,), jnp.float32),
            pltpu.VMEM((32,), jnp.float32),
            pltpu.VMEM((_Q,), jnp.int32),
            pltpu.VMEM((_Q,), jnp.int32),
            pltpu.VMEM((_BATCH,), jnp.float32),
            pltpu.VMEM_SHARED((2, _BATCH), jnp.int32),
            pltpu.SemaphoreType.DMA,
            pltpu.SemaphoreType.DMA,
            pltpu.SemaphoreType.DMA,
            pltpu.SemaphoreType.DMA,
            pltpu.SemaphoreType.DMA,
            pltpu.SemaphoreType.DMA,
        ],
        compiler_params=pltpu.CompilerParams(
            use_tc_tiling_on_sc=True, needs_layout_passes=False),
    )
    return f(x_t, tables_t).T
